# Initial kernel scaffold; baseline (speedup 1.0000x reference)
import jax
import jax.numpy as jnp
from jax.experimental import pallas as pl


def kernel(inputs, tables):
    raise NotImplementedError("write your pallas kernel here")


# speedup vs baseline: 1.0395x; 1.0395x over previous
import functools

import jax
import jax.numpy as jnp
from jax import lax
from jax.experimental import pallas as pl
from jax.experimental.pallas import tpu as pltpu
from jax.experimental.pallas import tpu_sc as plsc

N_FIELDS = 26
VOCAB = 100000
K = 32
BATCH = 4096

NC = 2
NS = 16
NW = NC * NS
LANES = 16

B_FLAT = BATCH * N_FIELDS
PER_W = B_FLAT // NW
CHUNK = 128
N_CHUNKS = PER_W // CHUNK

_mesh = plsc.VectorSubcoreMesh(core_axis_name="c", subcore_axis_name="s")


@functools.partial(
    pl.kernel,
    mesh=_mesh,
    out_type=jax.ShapeDtypeStruct((B_FLAT, K), jnp.float32),
    compiler_params=pltpu.CompilerParams(use_tc_tiling_on_sc=False),
    scratch_types=[
        pltpu.VMEM((N_CHUNKS, CHUNK), jnp.int32),
        pltpu.VMEM((N_CHUNKS, CHUNK), jnp.int32),
        pltpu.VMEM((PER_W, K), jnp.float32),
        pltpu.SemaphoreType.DMA,
    ],
)
def _embed_gather(idx_hbm, off_hbm, tab_hbm, out_hbm, idx_v, off_v, rows_v, sem):
    wid = lax.axis_index("s") * NC + lax.axis_index("c")

    pltpu.sync_copy(idx_hbm.at[wid], idx_v)
    pltpu.sync_copy(off_hbm, off_v)

    def _add_row(j, _):
        for c in range(CHUNK // LANES):
            sl = pl.ds(c * LANES, LANES)
            idx_v[j, sl] = idx_v[j, sl] + off_v[j, sl]
        return 0

    lax.fori_loop(0, N_CHUNKS, _add_row, 0)

    copies = [
        pltpu.async_copy(
            tab_hbm.at[idx_v.at[j]],
            rows_v.at[pl.ds(j * CHUNK, CHUNK)],
            sem,
        )
        for j in range(N_CHUNKS)
    ]
    for c in copies:
        c.wait()

    pltpu.sync_copy(rows_v, out_hbm.at[pl.ds(wid * PER_W, PER_W)])


def kernel(inputs, tables):
    idx = inputs.astype(jnp.int32).reshape(NW, N_CHUNKS, CHUNK)
    tab = tables.reshape(N_FIELDS * VOCAB, K)
    off = jnp.tile(
        jnp.arange(N_FIELDS, dtype=jnp.int32) * VOCAB, PER_W // N_FIELDS
    ).reshape(N_CHUNKS, CHUNK)
    out = _embed_gather(idx, off, tab)
    return out.reshape(BATCH, N_FIELDS * K)


# speedup vs baseline: 1.0407x; 1.0011x over previous
import functools

import jax
import jax.numpy as jnp
from jax import lax
from jax.experimental import pallas as pl
from jax.experimental.pallas import tpu as pltpu
from jax.experimental.pallas import tpu_sc as plsc

N_FIELDS = 26
VOCAB = 100000
K = 32
BATCH = 4096

NC = 2
NS = 16
NW = NC * NS
BPW = BATCH // NW

_mesh = plsc.VectorSubcoreMesh(core_axis_name="c", subcore_axis_name="s")


@functools.partial(
    pl.kernel,
    mesh=_mesh,
    out_type=jax.ShapeDtypeStruct((BATCH, N_FIELDS * K), jnp.float32),
    compiler_params=pltpu.CompilerParams(use_tc_tiling_on_sc=False),
    scratch_types=[
        pltpu.VMEM((N_FIELDS, BPW), jnp.int32),
        pltpu.VMEM((N_FIELDS, BPW, K), jnp.float32),
        pltpu.SemaphoreType.DMA,
        pltpu.SemaphoreType.DMA,
    ],
)
def _embed_gather(idx_hbm, tab_hbm, out_hbm, idx_v, rows_v, gsem, wsem):
    wid = lax.axis_index("s") * NC + lax.axis_index("c")
    base = wid * BPW

    pltpu.sync_copy(idx_hbm.at[:, pl.ds(base, BPW)], idx_v)

    gathers = [
        pltpu.async_copy(
            tab_hbm.at[f].at[idx_v.at[f]],
            rows_v.at[f],
            gsem,
        )
        for f in range(N_FIELDS)
    ]
    for g in gathers:
        g.wait()

    writes = [
        pltpu.async_copy(
            rows_v.at[f],
            out_hbm.at[pl.ds(base, BPW), pl.ds(f * K, K)],
            wsem,
        )
        for f in range(N_FIELDS)
    ]
    for w in writes:
        w.wait()


def kernel(inputs, tables):
    idx_t = inputs.astype(jnp.int32).T
    return _embed_gather(idx_t, tables)


# speedup vs baseline: 1.8212x; 1.7501x over previous
import functools

import jax
import jax.numpy as jnp
from jax import lax
from jax.experimental import pallas as pl
from jax.experimental.pallas import tpu as pltpu
from jax.experimental.pallas import tpu_sc as plsc

N_FIELDS = 26
VOCAB = 100000
K = 32
BATCH = 4096

NC = 2
NS = 16
NW = NC * NS
LANES = 16

R_TOTAL = N_FIELDS * K
R_PER_W = R_TOTAL // NW

_mesh = plsc.VectorSubcoreMesh(core_axis_name="c", subcore_axis_name="s")


@functools.partial(
    pl.kernel,
    mesh=_mesh,
    out_type=jax.ShapeDtypeStruct((R_TOTAL, BATCH), jnp.float32),
    compiler_params=pltpu.CompilerParams(use_tc_tiling_on_sc=False),
    scratch_types=[
        pltpu.VMEM((BATCH,), jnp.int32),
        pltpu.VMEM((BATCH,), jnp.int32),
        pltpu.VMEM((BATCH,), jnp.float32),
        pltpu.SemaphoreType.DMA,
    ],
)
def _embed_gather(idx_hbm, tab_hbm, out_hbm, v_v, gi_v, row_v, sem):
    wid = lax.axis_index("s") * NC + lax.axis_index("c")
    r0 = wid * R_PER_W

    def _row(j, _):
        r = r0 + j
        f = lax.div(r, K)
        pltpu.sync_copy(idx_hbm.at[f], v_v)
        base = r * VOCAB

        def _off(c, _):
            sl = pl.ds(c * LANES, LANES)
            gi_v[sl] = v_v[sl] + base
            return 0

        lax.fori_loop(0, BATCH // LANES, _off, 0)
        pltpu.async_copy(tab_hbm.at[gi_v], row_v, sem).wait()
        pltpu.sync_copy(row_v, out_hbm.at[r])
        return 0

    lax.fori_loop(0, R_PER_W, _row, 0)


def kernel(inputs, tables):
    idx_t = inputs.astype(jnp.int32).T
    tab_lin = jnp.transpose(tables, (0, 2, 1)).reshape(-1)
    out_t = _embed_gather(idx_t, tab_lin)
    return out_t.T


# speedup vs baseline: 1.9526x; 1.0721x over previous
import functools

import jax
import jax.numpy as jnp
from jax import lax
from jax.experimental import pallas as pl
from jax.experimental.pallas import tpu as pltpu
from jax.experimental.pallas import tpu_sc as plsc

N_FIELDS = 26
VOCAB = 100000
K = 32
BATCH = 4096

NC = 2
NS = 16
NW = NC * NS
LANES = 16

R_TOTAL = N_FIELDS * K
R_PER_W = R_TOTAL // NW

_mesh = plsc.VectorSubcoreMesh(core_axis_name="c", subcore_axis_name="s")


@functools.partial(
    pl.kernel,
    mesh=_mesh,
    out_type=jax.ShapeDtypeStruct((R_TOTAL, BATCH), jnp.float32),
    compiler_params=pltpu.CompilerParams(use_tc_tiling_on_sc=False),
    scratch_types=[
        pltpu.VMEM((BATCH,), jnp.int32),
        pltpu.VMEM((2, BATCH), jnp.int32),
        pltpu.VMEM((2, BATCH), jnp.float32),
        pltpu.SemaphoreType.DMA,
        pltpu.SemaphoreType.DMA,
    ],
)
def _embed_gather(idx_hbm, tab_hbm, out_hbm, v_v, gi_v, row_v, gsem, wsem):
    wid = lax.axis_index("s") * NC + lax.axis_index("c")
    r0 = wid * R_PER_W

    def _stage(j):
        r = r0 + j
        f = lax.div(r, K)
        pltpu.sync_copy(idx_hbm.at[f], v_v)
        base = r * VOCAB
        p = j % 2

        def _off(c, _):
            sl = pl.ds(c * LANES, LANES)
            gi_v[p, sl] = v_v[sl] + base
            return 0

        lax.fori_loop(0, BATCH // LANES, _off, 0)

    def _gather(j):
        p = j % 2
        return pltpu.async_copy(tab_hbm.at[gi_v.at[p]], row_v.at[p], gsem)

    def _write(j):
        p = j % 2
        return pltpu.async_copy(row_v.at[p], out_hbm.at[r0 + j], wsem)

    _stage(0)
    g = _gather(0)
    w_prev = None
    for j in range(R_PER_W):
        if j + 1 < R_PER_W:
            _stage(j + 1)
        g.wait()
        if j + 1 < R_PER_W:
            if w_prev is not None:
                w_prev.wait()
            g = _gather(j + 1)
        w = _write(j)
        w_prev = w
    w_prev.wait()


def kernel(inputs, tables):
    idx_t = inputs.astype(jnp.int32).T
    tab_lin = jnp.transpose(tables, (0, 2, 1)).reshape(-1)
    out_t = _embed_gather(idx_t, tab_lin)
    return out_t.T
